# baseline (speedup 1.0000x reference)
import functools

import jax
import jax.numpy as jnp
from jax import lax
from jax.experimental import pallas as pl
from jax.experimental.pallas import tpu as pltpu
from jax.experimental.pallas import tpu_sc as plsc

VOCAB = 457
EMB = 64
BATCH = 1024
SEQ = 457

_info = plsc.get_sparse_core_info()
_NC = _info.num_cores
_NS = _info.num_subcores
NW = _NC * _NS

NFULL = SEQ // NW
NEXTRA = SEQ - NFULL * NW
PIECE = 8 * 1024

_mesh = plsc.VectorSubcoreMesh(core_axis_name="c", subcore_axis_name="s")


@functools.partial(
    pl.kernel,
    mesh=_mesh,
    out_type=(
        jax.ShapeDtypeStruct((SEQ, 8, 8, 8, 128), jnp.float32),
        jax.ShapeDtypeStruct((SEQ, 8, 8, 8, 128), jnp.float32),
    ),
    scratch_types=[
        pltpu.VMEM((VOCAB * EMB,), jnp.float32),
        pltpu.VMEM((BATCH,), jnp.int32),
        pltpu.VMEM((BATCH,), jnp.int32),
        pltpu.VMEM((8, 8, 128), jnp.float32),
        pltpu.VMEM((8, 8, 128), jnp.float32),
        pltpu.SemaphoreType.DMA,
        pltpu.SemaphoreType.DMA,
        pltpu.SemaphoreType.DMA,
        pltpu.SemaphoreType.DMA,
    ],
    compiler_params=pltpu.CompilerParams(
        use_tc_tiling_on_sc=False, needs_layout_passes=False),
)
def _lookup(seq_t, exp_t, w_flat, o1, o2,
            table_v, ir0, ir1, b0, b1, is0, is1, ss0, ss1):
    wid = lax.axis_index("s") * _NC + lax.axis_index("c")

    pltpu.sync_copy(w_flat, table_v)

    irs = (ir0, ir1)
    bufs = (b0, b1)
    isems = (is0, is1)
    ssems = (ss0, ss1)

    def compute_piece(ir, et, buf):
        def gbody(g, c):
            iv = ir[pl.ds(g * 16, 16)]
            base = iv * EMB + (et * 8)
            bt = g >> 3
            boff = (g & 7) * 16
            for e8 in range(8):
                v = plsc.load_gather(table_v, [base + e8])
                buf[bt, e8, pl.ds(boff, 16)] = v
            return c
        lax.fori_loop(0, 64, gbody, 0)

    def wait_sc(b):
        pltpu.make_async_copy(bufs[b], o1.at[0, 0], ssems[b]).wait()

    def run(idx_t, out_hbm, first):
        def load_idx(i, b):
            pltpu.async_copy(idx_t.at[wid + i * NW], irs[b], isems[b])

        def wait_idx(b):
            pltpu.make_async_copy(idx_t.at[0], irs[b], isems[b]).wait()

        def do_s(i, ib, first_s=False):
            for et in range(8):
                bb = et & 1
                if not (first_s and et < 2):
                    wait_sc(bb)
                compute_piece(irs[ib], et, bufs[bb])
                pltpu.async_copy(
                    bufs[bb], out_hbm.at[wid + i * NW, et], ssems[bb])

        load_idx(0, 0)
        wait_idx(0)
        load_idx(1, 1)
        do_s(0, 0, first_s=first)
        wait_idx(1)
        load_idx(2, 0)
        do_s(1, 1)

        def body(k, c):
            i = 2 * k
            wait_idx(0)
            load_idx(i + 1, 1)
            do_s(i, 0)
            wait_idx(1)

            @pl.when(wid + (i + 2) * NW < SEQ)
            def _():
                load_idx(i + 2, 0)

            do_s(i + 1, 1)
            return c

        lax.fori_loop(1, NFULL // 2, body, 0)

        @pl.when(wid < NEXTRA)
        def _():
            wait_idx(0)
            do_s(NFULL, 0)

    run(seq_t, o1, True)
    run(exp_t, o2, False)
    wait_sc(0)
    wait_sc(1)


def kernel(seqs, exps, W):
    p1, p2 = _lookup(seqs.T, exps.T, W.reshape(-1))

    def unpack(p):
        return p.transpose(2, 4, 0, 1, 3).reshape(BATCH, SEQ, EMB)

    return unpack(p1), unpack(p2)


# speedup vs baseline: 1.6582x; 1.6582x over previous
import functools

import jax
import jax.numpy as jnp
from jax import lax
from jax.experimental import pallas as pl
from jax.experimental.pallas import tpu as pltpu
from jax.experimental.pallas import tpu_sc as plsc

VOCAB = 457
EMB = 64
BATCH = 1024
SEQ = 457

_info = plsc.get_sparse_core_info()
_NC = _info.num_cores
_NS = _info.num_subcores
NW = _NC * _NS

MAXS = SEQ // NW + 1

_mesh = plsc.VectorSubcoreMesh(core_axis_name="c", subcore_axis_name="s")


@functools.partial(
    pl.kernel,
    mesh=_mesh,
    out_type=(
        jax.ShapeDtypeStruct((SEQ, 8, 8, 8, 128), jnp.float32),
        jax.ShapeDtypeStruct((SEQ, 8, 8, 8, 128), jnp.float32),
    ),
    scratch_types=[
        pltpu.VMEM((VOCAB * EMB,), jnp.float32),
        pltpu.VMEM((MAXS * BATCH,), jnp.int32),
        pltpu.VMEM((8, 8, 128), jnp.float32),
        pltpu.VMEM((8, 8, 128), jnp.float32),
        pltpu.SemaphoreType.DMA,
        pltpu.SemaphoreType.DMA,
    ],
    compiler_params=pltpu.CompilerParams(
        use_tc_tiling_on_sc=False, needs_layout_passes=False),
)
def _lookup(seq_f, exp_f, w_flat, o1, o2, table_v, ir_all, b0, b1, ss0, ss1):
    wid = lax.axis_index("s") * _NC + lax.axis_index("c")
    s0 = (wid * SEQ) >> 5
    n = (((wid + 1) * SEQ) >> 5) - s0

    pltpu.sync_copy(w_flat, table_v)

    bufs = (b0, b1)
    ssems = (ss0, ss1)

    def compute_piece(ioff, et, buf):
        @plsc.parallel_loop(0, 64, unroll=4)
        def gbody(g):
            iv = ir_all[pl.ds(ioff + g * 16, 16)]
            base = iv * EMB + (et * 8)
            bt = g >> 3
            boff = (g & 7) * 16
            for e8 in range(8):
                v = plsc.load_gather(table_v, [base + e8])
                buf[bt, e8, pl.ds(boff, 16)] = v

    def wait_sc(b):
        pltpu.make_async_copy(bufs[b], o1.at[0, 0], ssems[b]).wait()

    def do_s(i, out_hbm, first_s=False):
        for et in range(8):
            bb = et & 1
            if not (first_s and et < 2):
                wait_sc(bb)
            compute_piece(i * BATCH, et, bufs[bb])
            pltpu.async_copy(bufs[bb], out_hbm.at[s0 + i, et], ssems[bb])

    def run(idx_f, out_hbm, first):
        pltpu.sync_copy(
            idx_f.at[pl.ds(s0 * BATCH, MAXS * BATCH)], ir_all)
        lo = 0
        if first:
            do_s(0, out_hbm, first_s=True)
            lo = 1

        def body(i, c):
            do_s(i, out_hbm)
            return c

        lax.fori_loop(lo, n, body, 0)

    run(seq_f, o1, True)
    run(exp_f, o2, False)
    wait_sc(0)
    wait_sc(1)


def kernel(seqs, exps, W):
    p1, p2 = _lookup(
        seqs.T.reshape(-1), exps.T.reshape(-1), W.reshape(-1))

    def unpack(p):
        return p.transpose(2, 4, 0, 1, 3).reshape(BATCH, SEQ, EMB)

    return unpack(p1), unpack(p2)
